# baseline (speedup 1.0000x reference)
import functools

import jax
import jax.numpy as jnp
from jax import lax
from jax.experimental import pallas as pl
from jax.experimental.pallas import tpu as pltpu
from jax.experimental.pallas import tpu_sc as plsc

VOCAB = 100000
MAX_LEN = 50
EMB = 300
BATCH = 1024

_SUBW = 16
_NSUB = VOCAB * EMB // _SUBW
_WIN = 20

_info = plsc.get_sparse_core_info()
_NW = _info.num_cores * _info.num_subcores
_RPW = BATCH // _NW
_IDX_N = MAX_LEN * _WIN
_GATHERS = -(-_IDX_N // 128)
_ROW_F32 = MAX_LEN * EMB
_PB = 16


def _sc_embed():
  mesh = plsc.VectorSubcoreMesh(core_axis_name="c", subcore_axis_name="s")
  out_sds = jax.ShapeDtypeStruct((BATCH * MAX_LEN * EMB,), jnp.float32)

  @functools.partial(
      pl.kernel,
      mesh=mesh,
      out_type=(out_sds, out_sds, out_sds, out_sds),
      compiler_params=pltpu.CompilerParams(use_tc_tiling_on_sc=False),
      scratch_types=[
          pltpu.VMEM((_RPW * MAX_LEN + 16,), jnp.int32),
          pltpu.VMEM((_RPW + 16,), jnp.int32),
          pltpu.VMEM((_IDX_N + 24,), jnp.int32),
          pltpu.VMEM((_IDX_N + 24,), jnp.int32),
          pltpu.VMEM((_IDX_N + 8, _SUBW), jnp.float32),
          pltpu.VMEM((_IDX_N + 8, _SUBW), jnp.float32),
          pltpu.VMEM((_PB + _ROW_F32 + 32,), jnp.float32),
          pltpu.VMEM((_PB + _ROW_F32 + 32,), jnp.float32),
          pltpu.SemaphoreType.DMA,
          pltpu.SemaphoreType.DMA,
          pltpu.SemaphoreType.DMA,
          pltpu.SemaphoreType.DMA,
      ],
  )
  def k(jd, jr, we, pe, jdl, jrl, wel, pel, tsub,
        o0, o1, o2, o3, idxc_v, lens_v, idxb0, idxb1, win0, win1,
        pk0, pk1, sg0, sg1, sw0, sw1):
    wid = lax.axis_index("s") * _info.num_cores + lax.axis_index("c")
    b0 = wid * _RPW
    iota = lax.iota(jnp.int32, 16)
    iota4 = iota + jnp.full((16,), 4, jnp.int32)
    nsub1 = jnp.full((16,), _NSUB - 1, jnp.int32)
    zero = jnp.zeros((16,), jnp.float32)

    def mk_row(bl, ib):
      def mk(c, _):
        for j in range(5):
          p = 5 * c + j
          t = idxc_v[pl.ds(bl * MAX_LEN + p, 16)][0]
          s = (t * 75) // 4
          sv = jnp.full((16,), s, jnp.int32)
          ib[pl.ds(_WIN * p, 16)] = jnp.minimum(sv + iota, nsub1)
          ib[pl.ds(_WIN * p + 4, 16)] = jnp.minimum(sv + iota4, nsub1)
        return 0

      lax.fori_loop(0, MAX_LEN // 5, mk, 0)

    def gather_copies(ib, wv, sg):
      for g in range(_GATHERS):
        cnt = min(128, _IDX_N - 128 * g)
        yield pltpu.make_async_copy(
            tsub.at[ib.at[pl.ds(128 * g, cnt)]],
            wv.at[pl.ds(128 * g, cnt)], sg)

    def fire_gathers(ib, wv, sg):
      for cp in gather_copies(ib, wv, sg):
        cp.start()

    def wait_gathers(ib, wv, sg):
      for cp in gather_copies(ib, wv, sg):
        cp.wait()

    def comp_row(bl, wv, pk):
      len_s = lens_v[pl.ds(bl, 16)][0]

      def comp(c, _):
        lv = jnp.full((16,), len_s, jnp.int32)
        for j in range(5):
          p = 5 * c + j
          t = idxc_v[pl.ds(bl * MAX_LEN + p, 16)][0]
          s = (t * 75) // 4
          o = t * EMB - s * _SUBW
          ov = jnp.full((16,), o, jnp.int32)
          d = _PB + EMB * p - o
          f_len = jnp.where(jnp.full((16,), p, jnp.int32) < lv,
                            jnp.full((16,), 1.0, jnp.float32), zero)
          m_head = iota >= ov
          m_t18 = iota < ov + jnp.full((16,), 12, jnp.int32)
          m_t19 = iota < ov - jnp.full((16,), 4, jnp.int32)
          for i in range(_WIN):
            v = wv[_WIN * p + i, pl.ds(0, 16)] * f_len
            if i == 0:
              v = jnp.where(m_head, v, pk[pl.ds(d, 16)])
            elif i == 18:
              v = jnp.where(m_t18, v, pk[pl.ds(d + 16 * i, 16)])
            elif i == 19:
              v = jnp.where(m_t19, v, pk[pl.ds(d + 16 * i, 16)])
            pk[pl.ds(d + 16 * i, 16)] = v
        return 0

      lax.fori_loop(0, MAX_LEN // 5, comp, 0)

    slots = ((idxb0, win0, pk0, sg0, sw0), (idxb1, win1, pk1, sg1, sw1))

    for idx_hbm, len_hbm, out_hbm in ((jd, jdl, o0), (jr, jrl, o1),
                                      (we, wel, o2), (pe, pel, o3)):
      pltpu.sync_copy(idx_hbm.at[pl.ds(b0 * MAX_LEN, _RPW * MAX_LEN)],
                      idxc_v.at[pl.ds(0, _RPW * MAX_LEN)])
      pltpu.sync_copy(len_hbm.at[pl.ds(b0, _RPW)],
                      lens_v.at[pl.ds(0, _RPW)])

      def wb_copy(bl, pk, sw, out_hbm=out_hbm):
        return pltpu.make_async_copy(
            pk.at[pl.ds(_PB, _ROW_F32)],
            out_hbm.at[pl.ds((b0 + bl) * _ROW_F32, _ROW_F32)], sw)

      mk_row(0, idxb0)
      fire_gathers(idxb0, win0, sg0)

      def step(jj, _, wb_copy=wb_copy):
        for par, (ib, wv, pk, sg, sw) in enumerate(slots):
          ibn, wvn, _, sgn, _ = slots[1 - par]
          bl = 2 * jj + par

          @pl.when(bl < _RPW - 1)
          def _():
            mk_row(bl + 1, ibn)
            fire_gathers(ibn, wvn, sgn)

          wait_gathers(ib, wv, sg)

          @pl.when(bl >= 2)
          def _():
            wb_copy(bl - 2, pk, sw).wait()

          comp_row(bl, wv, pk)
          wb_copy(bl, pk, sw).start()
        return 0

      lax.fori_loop(0, _RPW // 2, step, 0)
      wb_copy(_RPW - 2, pk0, sw0).wait()
      wb_copy(_RPW - 1, pk1, sw1).wait()

  return k


def kernel(jobduty, jobreq, wrokexp, projexp,
           jobduty_len, jobreq_len, wrokexp_len, projexp_len,
           w2v_table):
  f = _sc_embed()
  tsub = w2v_table.reshape(_NSUB, _SUBW)
  outs = f(jobduty.reshape(-1), jobreq.reshape(-1),
           wrokexp.reshape(-1), projexp.reshape(-1),
           jobduty_len, jobreq_len, wrokexp_len, projexp_len, tsub)
  return tuple(o.reshape(BATCH, MAX_LEN, EMB) for o in outs)


# speedup vs baseline: 1.2396x; 1.2396x over previous
import functools

import jax
import jax.numpy as jnp
from jax import lax
from jax.experimental import pallas as pl
from jax.experimental.pallas import tpu as pltpu
from jax.experimental.pallas import tpu_sc as plsc

VOCAB = 100000
MAX_LEN = 50
EMB = 300
BATCH = 1024

_SUBW = 16
_NSUB = VOCAB * EMB // _SUBW
_WIN = 20

_info = plsc.get_sparse_core_info()
_NW = _info.num_cores * _info.num_subcores
_RPW = BATCH // _NW
_IDX_N = MAX_LEN * _WIN
_GATHERS = -(-_IDX_N // 128)
_ROW_F32 = MAX_LEN * EMB
_PB = 16


def _sc_embed():
  mesh = plsc.VectorSubcoreMesh(core_axis_name="c", subcore_axis_name="s")
  out_sds = jax.ShapeDtypeStruct((BATCH * MAX_LEN * EMB,), jnp.float32)

  @functools.partial(
      pl.kernel,
      mesh=mesh,
      out_type=(out_sds, out_sds, out_sds, out_sds),
      compiler_params=pltpu.CompilerParams(use_tc_tiling_on_sc=False),
      scratch_types=[
          pltpu.VMEM((_RPW * MAX_LEN + 16,), jnp.int32),
          pltpu.VMEM((_RPW + 16,), jnp.int32),
          pltpu.VMEM((_IDX_N,), jnp.int32),
          pltpu.VMEM((_IDX_N,), jnp.int32),
          pltpu.VMEM((_IDX_N, _SUBW), jnp.float32),
          pltpu.VMEM((_IDX_N, _SUBW), jnp.float32),
          pltpu.VMEM((_PB + _ROW_F32 + 32,), jnp.float32),
          pltpu.VMEM((_PB + _ROW_F32 + 32,), jnp.float32),
          pltpu.SemaphoreType.DMA,
          pltpu.SemaphoreType.DMA,
          pltpu.SemaphoreType.DMA,
          pltpu.SemaphoreType.DMA,
          pltpu.SemaphoreType.DMA,
          pltpu.SemaphoreType.DMA,
      ],
  )
  def k(jd, jr, we, pe, jdl, jrl, wel, pel, tsub,
        wjd, wjr, wwe, wpe,
        o0, o1, o2, o3, idxc_v, lens_v, idxb0, idxb1, win0, win1,
        pk0, pk1, si0, si1, sg0, sg1, sw0, sw1):
    wid = lax.axis_index("s") * _info.num_cores + lax.axis_index("c")
    b0 = wid * _RPW
    iota = lax.iota(jnp.int32, 16)
    zero = jnp.zeros((16,), jnp.float32)

    def gather_copies(ib, wv, sg):
      for g in range(_GATHERS):
        cnt = min(128, _IDX_N - 128 * g)
        yield pltpu.make_async_copy(
            tsub.at[ib.at[pl.ds(128 * g, cnt)]],
            wv.at[pl.ds(128 * g, cnt)], sg)

    def fire_gathers(ib, wv, sg):
      for cp in gather_copies(ib, wv, sg):
        cp.start()

    def wait_gathers(ib, wv, sg):
      for cp in gather_copies(ib, wv, sg):
        cp.wait()

    def comp_row(bl, wv, pk):
      len_s = lens_v[pl.ds(bl, 16)][0]

      def comp(p, _):
        @pl.when(p < len_s)
        def _():
          t = idxc_v[pl.ds(bl * MAX_LEN + p, 16)][0]
          s = (t * 75) // 4
          o = t * EMB - s * _SUBW
          ov = jnp.full((16,), o, jnp.int32)
          d = _PB + EMB * p - o
          m_head = iota >= ov
          m_t18 = iota < ov + jnp.full((16,), 12, jnp.int32)
          m_t19 = iota < ov - jnp.full((16,), 4, jnp.int32)
          for i in range(_WIN):
            v = wv[_WIN * p + i, pl.ds(0, 16)]
            if i == 0:
              v = jnp.where(m_head, v, pk[pl.ds(d, 16)])
            elif i == 18:
              v = jnp.where(m_t18, v, pk[pl.ds(d + 16 * i, 16)])
            elif i == 19:
              v = jnp.where(m_t19, v, pk[pl.ds(d + 16 * i, 16)])
            pk[pl.ds(d + 16 * i, 16)] = v

        @pl.when(p >= len_s)
        def _():
          for off in tuple(range(0, EMB - 16, 16)) + (EMB - 16,):
            pk[pl.ds(_PB + EMB * p + off, 16)] = zero

        return 0

      lax.fori_loop(0, MAX_LEN, comp, 0)

    slots = ((idxb0, win0, pk0, si0, sg0, sw0),
             (idxb1, win1, pk1, si1, sg1, sw1))

    for idx_hbm, len_hbm, widx_hbm, out_hbm in (
        (jd, jdl, wjd, o0), (jr, jrl, wjr, o1),
        (we, wel, wwe, o2), (pe, pel, wpe, o3)):
      pltpu.sync_copy(idx_hbm.at[pl.ds(b0 * MAX_LEN, _RPW * MAX_LEN)],
                      idxc_v.at[pl.ds(0, _RPW * MAX_LEN)])
      pltpu.sync_copy(len_hbm.at[pl.ds(b0, _RPW)],
                      lens_v.at[pl.ds(0, _RPW)])

      def il_copy(bl, ib, si, widx_hbm=widx_hbm):
        return pltpu.make_async_copy(
            widx_hbm.at[pl.ds((b0 + bl) * _IDX_N, _IDX_N)], ib, si)

      def wb_copy(bl, pk, sw, out_hbm=out_hbm):
        return pltpu.make_async_copy(
            pk.at[pl.ds(_PB, _ROW_F32)],
            out_hbm.at[pl.ds((b0 + bl) * _ROW_F32, _ROW_F32)], sw)

      il_copy(0, idxb0, si0).start()
      il_copy(0, idxb0, si0).wait()
      fire_gathers(idxb0, win0, sg0)
      il_copy(1, idxb1, si1).start()

      def step(jj, _, il_copy=il_copy, wb_copy=wb_copy):
        for par, (ib, wv, pk, si, sg, sw) in enumerate(slots):
          ibn, wvn, _, sin, sgn, _ = slots[1 - par]
          bl = 2 * jj + par

          @pl.when(bl < _RPW - 1)
          def _():
            il_copy(bl + 1, ibn, sin).wait()
            fire_gathers(ibn, wvn, sgn)

          wait_gathers(ib, wv, sg)

          @pl.when(bl < _RPW - 2)
          def _():
            il_copy(bl + 2, ib, si).start()

          @pl.when(bl >= 2)
          def _():
            wb_copy(bl - 2, pk, sw).wait()

          comp_row(bl, wv, pk)
          wb_copy(bl, pk, sw).start()
        return 0

      lax.fori_loop(0, _RPW // 2, step, 0)
      wb_copy(_RPW - 2, pk0, sw0).wait()
      wb_copy(_RPW - 1, pk1, sw1).wait()

  return k


def kernel(jobduty, jobreq, wrokexp, projexp,
           jobduty_len, jobreq_len, wrokexp_len, projexp_len,
           w2v_table):
  f = _sc_embed()
  tsub = w2v_table.reshape(_NSUB, _SUBW)

  win_off = jnp.arange(_WIN, dtype=jnp.int32)[None, None, :]

  def widx(idx):
    s = (idx * 75) // 4
    return jnp.minimum(s[:, :, None] + win_off, _NSUB - 1).reshape(-1)

  outs = f(jobduty.reshape(-1), jobreq.reshape(-1),
           wrokexp.reshape(-1), projexp.reshape(-1),
           jobduty_len, jobreq_len, wrokexp_len, projexp_len, tsub,
           widx(jobduty), widx(jobreq), widx(wrokexp), widx(projexp))
  return tuple(o.reshape(BATCH, MAX_LEN, EMB) for o in outs)


# speedup vs baseline: 1.2663x; 1.0215x over previous
import functools

import jax
import jax.numpy as jnp
from jax import lax
from jax.experimental import pallas as pl
from jax.experimental.pallas import tpu as pltpu
from jax.experimental.pallas import tpu_sc as plsc

VOCAB = 100000
MAX_LEN = 50
EMB = 300
BATCH = 1024

_SUBW = 16
_NSUB = VOCAB * EMB // _SUBW
_WIN = 20

_info = plsc.get_sparse_core_info()
_NW = _info.num_cores * _info.num_subcores
_RPW = BATCH // _NW
_IDX_N = MAX_LEN * _WIN
_GATHERS = -(-_IDX_N // 128)
_ROW_F32 = MAX_LEN * EMB
_PB = 16


def _sc_embed():
  mesh = plsc.VectorSubcoreMesh(core_axis_name="c", subcore_axis_name="s")
  out_sds = jax.ShapeDtypeStruct((BATCH * MAX_LEN * EMB,), jnp.float32)

  @functools.partial(
      pl.kernel,
      mesh=mesh,
      out_type=(out_sds, out_sds, out_sds, out_sds),
      compiler_params=pltpu.CompilerParams(use_tc_tiling_on_sc=False),
      scratch_types=[
          pltpu.VMEM((_RPW * MAX_LEN + 16,), jnp.int32),
          pltpu.VMEM((_RPW + 16,), jnp.int32),
          pltpu.VMEM((_IDX_N,), jnp.int32),
          pltpu.VMEM((_IDX_N,), jnp.int32),
          pltpu.VMEM((_IDX_N, _SUBW), jnp.float32),
          pltpu.VMEM((_IDX_N, _SUBW), jnp.float32),
          pltpu.VMEM((_PB + _ROW_F32 + 32,), jnp.float32),
          pltpu.VMEM((_PB + _ROW_F32 + 32,), jnp.float32),
          pltpu.SemaphoreType.DMA,
          pltpu.SemaphoreType.DMA,
          pltpu.SemaphoreType.DMA,
          pltpu.SemaphoreType.DMA,
          pltpu.SemaphoreType.DMA,
          pltpu.SemaphoreType.DMA,
      ],
  )
  def k(jd, jr, we, pe, jdl, jrl, wel, pel, tsub,
        wjd, wjr, wwe, wpe,
        o0, o1, o2, o3, idxc_v, lens_v, idxb0, idxb1, win0, win1,
        pk0, pk1, si0, si1, sg0, sg1, sw0, sw1):
    wid = lax.axis_index("s") * _info.num_cores + lax.axis_index("c")
    b0 = wid * _RPW
    iota = lax.iota(jnp.int32, 16)
    zero = jnp.zeros((16,), jnp.float32)

    def gather_copies(ib, wv, sg):
      for g in range(_GATHERS):
        cnt = min(128, _IDX_N - 128 * g)
        yield pltpu.make_async_copy(
            tsub.at[ib.at[pl.ds(128 * g, cnt)]],
            wv.at[pl.ds(128 * g, cnt)], sg)

    def fire_gathers(ib, wv, sg):
      for cp in gather_copies(ib, wv, sg):
        cp.start()

    def wait_gathers(ib, wv, sg):
      for cp in gather_copies(ib, wv, sg):
        cp.wait()

    def comp_row(bl, wv, pk):
      len_s = lens_v[pl.ds(bl, 16)][0]

      def comp_valid(p, _):
        t = idxc_v[pl.ds(bl * MAX_LEN + p, 16)][0]
        s = (t * 75) // 4
        o = t * EMB - s * _SUBW
        ov = jnp.full((16,), o, jnp.int32)
        d = _PB + EMB * p - o
        m_head = iota >= ov
        m_t18 = iota < ov + jnp.full((16,), 12, jnp.int32)
        m_t19 = iota < ov - jnp.full((16,), 4, jnp.int32)
        for i in range(_WIN):
          v = wv[_WIN * p + i, pl.ds(0, 16)]
          if i == 0:
            v = jnp.where(m_head, v, pk[pl.ds(d, 16)])
          elif i == 18:
            v = jnp.where(m_t18, v, pk[pl.ds(d + 16 * i, 16)])
          elif i == 19:
            v = jnp.where(m_t19, v, pk[pl.ds(d + 16 * i, 16)])
          pk[pl.ds(d + 16 * i, 16)] = v
        return 0

      def comp_zero(p, _):
        for off in tuple(range(0, EMB - 16, 16)) + (EMB - 16,):
          pk[pl.ds(_PB + EMB * p + off, 16)] = zero
        return 0

      lax.fori_loop(0, len_s, comp_valid, 0)
      lax.fori_loop(len_s, MAX_LEN, comp_zero, 0)

    slots = ((idxb0, win0, pk0, si0, sg0, sw0),
             (idxb1, win1, pk1, si1, sg1, sw1))

    for idx_hbm, len_hbm, widx_hbm, out_hbm in (
        (jd, jdl, wjd, o0), (jr, jrl, wjr, o1),
        (we, wel, wwe, o2), (pe, pel, wpe, o3)):
      pltpu.sync_copy(idx_hbm.at[pl.ds(b0 * MAX_LEN, _RPW * MAX_LEN)],
                      idxc_v.at[pl.ds(0, _RPW * MAX_LEN)])
      pltpu.sync_copy(len_hbm.at[pl.ds(b0, _RPW)],
                      lens_v.at[pl.ds(0, _RPW)])

      def il_copy(bl, ib, si, widx_hbm=widx_hbm):
        return pltpu.make_async_copy(
            widx_hbm.at[pl.ds((b0 + bl) * _IDX_N, _IDX_N)], ib, si)

      def wb_copy(bl, pk, sw, out_hbm=out_hbm):
        return pltpu.make_async_copy(
            pk.at[pl.ds(_PB, _ROW_F32)],
            out_hbm.at[pl.ds((b0 + bl) * _ROW_F32, _ROW_F32)], sw)

      il_copy(0, idxb0, si0).start()
      il_copy(0, idxb0, si0).wait()
      fire_gathers(idxb0, win0, sg0)
      il_copy(1, idxb1, si1).start()

      def step(jj, _, il_copy=il_copy, wb_copy=wb_copy):
        for par, (ib, wv, pk, si, sg, sw) in enumerate(slots):
          ibn, wvn, _, sin, sgn, _ = slots[1 - par]
          bl = 2 * jj + par

          @pl.when(bl < _RPW - 1)
          def _():
            il_copy(bl + 1, ibn, sin).wait()
            fire_gathers(ibn, wvn, sgn)

          wait_gathers(ib, wv, sg)

          @pl.when(bl < _RPW - 2)
          def _():
            il_copy(bl + 2, ib, si).start()

          @pl.when(bl >= 2)
          def _():
            wb_copy(bl - 2, pk, sw).wait()

          comp_row(bl, wv, pk)
          wb_copy(bl, pk, sw).start()
        return 0

      lax.fori_loop(0, _RPW // 2, step, 0)
      wb_copy(_RPW - 2, pk0, sw0).wait()
      wb_copy(_RPW - 1, pk1, sw1).wait()

  return k


def kernel(jobduty, jobreq, wrokexp, projexp,
           jobduty_len, jobreq_len, wrokexp_len, projexp_len,
           w2v_table):
  f = _sc_embed()
  tsub = w2v_table.reshape(_NSUB, _SUBW)

  win_off = jnp.arange(_WIN, dtype=jnp.int32)[None, None, :]

  def widx(idx):
    s = (idx * 75) // 4
    return jnp.minimum(s[:, :, None] + win_off, _NSUB - 1).reshape(-1)

  outs = f(jobduty.reshape(-1), jobreq.reshape(-1),
           wrokexp.reshape(-1), projexp.reshape(-1),
           jobduty_len, jobreq_len, wrokexp_len, projexp_len, tsub,
           widx(jobduty), widx(jobreq), widx(wrokexp), widx(projexp))
  return tuple(o.reshape(BATCH, MAX_LEN, EMB) for o in outs)
